# baseline (speedup 1.0000x reference)
import functools

import jax
import jax.numpy as jnp
from jax import lax
from jax.experimental import pallas as pl
from jax.experimental.pallas import tpu as pltpu
from jax.experimental.pallas import tpu_sc as plsc

K = 64
ROWS = 128
COLS = 32768
NITER = COLS // 16
CHUNK = 256
CAP = 4096
BUF = CAP + 16 * CHUNK + 16
BIG = 1 << 30
NEG_INF = float("-inf")

NC, NS = 2, 16
NW = NC * NS
RPW = ROWS // NW


def _body(logits_hbm, vals_hbm, idx_hbm,
          row_va, row_vb, cval, cidx, cval2, cidx2, outv, outi,
          sem_a, sem_b):
    lane = lax.iota(jnp.int32, 16)
    lane0 = lane == 0
    wid = lax.axis_index("s") * NC + lax.axis_index("c")

    def subset_min_threshold(src, nv):
        ninf = jnp.full((16,), NEG_INF)

        @plsc.parallel_loop(0, nv - (nv % 4), step=4, unroll=8,
                            carry=(ninf, ninf, ninf, ninf))
        def acc(j, c):
            a0, a1, a2, a3 = c
            a0 = jnp.maximum(a0, src[pl.ds(j * 16, 16)])
            a1 = jnp.maximum(a1, src[pl.ds((j + 1) * 16, 16)])
            a2 = jnp.maximum(a2, src[pl.ds((j + 2) * 16, 16)])
            a3 = jnp.maximum(a3, src[pl.ds((j + 3) * 16, 16)])
            return (a0, a1, a2, a3)

        a0, a1, a2, a3 = acc
        def leftover(j, a):
            return jnp.maximum(a, src[pl.ds(j * 16, 16)])

        a0 = lax.fori_loop(nv - (nv % 4), nv, leftover, a0)
        return jnp.min(jnp.minimum(jnp.minimum(a0, a1),
                                   jnp.minimum(a2, a3)))

    def filter_append(src_v, src_i, dval, didx, lo, hi, t, off, use_iota,
                      pipelined=True):
        offv = jnp.full((16,), jnp.int32(0)) + off

        def step(i, offv):
            v = src_v[pl.ds(i * 16, 16)]
            if use_iota:
                ix = i * 16 + lane
            else:
                ix = src_i[pl.ds(i * 16, 16)]
            mask = v >= t
            ones = mask.astype(jnp.int32)
            pos = offv + plsc.cumsum(ones) - 1
            if dval is not None:
                plsc.store_scatter(dval, [pos], v, mask=mask)
            plsc.store_scatter(didx, [pos], ix, mask=mask)
            return offv + plsc.all_reduce_population_count(mask)

        if pipelined:
            run = plsc.parallel_loop(lo, hi, unroll=4, carry=offv)(step)
        else:
            run = lax.fori_loop(lo, hi, step, offv)
        return jnp.max(run)

    def extract_topk(bval, bidx, off, obase=0):
        bval[pl.ds(off, 16)] = jnp.full((16,), NEG_INF)
        nv = (off + 15) // 16
        ninf = jnp.full((16,), NEG_INF)
        bigv = jnp.full((16,), jnp.int32(BIG))

        def ext_step(k, prev):
            m_prev, j_prev = prev

            @plsc.parallel_loop(0, nv, unroll=1, carry=(ninf, bigv))
            def scan(j, acc):
                av, ai = acc
                v = bval[pl.ds(j * 16, 16)]
                ix = bidx[pl.ds(j * 16, 16)]
                hit = (v == m_prev) & (ix == j_prev)
                v = jnp.where(hit, NEG_INF, v)
                bval[pl.ds(j * 16, 16)] = v
                take = (v > av) | ((v == av) & (ix < ai))
                return (jnp.where(take, v, av), jnp.where(take, ix, ai))

            av, ai = scan
            m = jnp.max(av)
            jm = jnp.min(jnp.where(av == m, ai, BIG))
            plsc.store_compressed(
                outv.at[pl.ds(obase + k, 16)], jnp.full((16,), m), mask=lane0)
            plsc.store_compressed(
                outi.at[pl.ds(obase + k, 16)], jnp.full((16,), jm), mask=lane0)
            return (m, jm)

        m, _ = lax.fori_loop(0, K, ext_step,
                             (jnp.float32(jnp.inf), jnp.int32(-1)))
        return m

    def row_body(r, buf):
        def gather_values(off):
            cidx[pl.ds(off, 16)] = jnp.full((16,), jnp.int32(0))
            nv = (off + 15) // 16

            def gv(j, _):
                ix = cidx[pl.ds(j * 16, 16)]
                cval[pl.ds(j * 16, 16)] = plsc.load_gather(buf, [ix])
                return 0

            lax.fori_loop(0, nv, gv, 0)

        def compact(carry):
            off, t = carry
            gather_values(off)
            t_new = extract_topk(cval, cidx, off, RPW * K)
            for j in range(K // 16):
                cidx[pl.ds(j * 16, 16)] = outi[pl.ds(RPW * K + j * 16, 16)]
            return (jnp.int32(K), t_new)

        t0 = subset_min_threshold(buf, NITER)

        def pb_chunk(c, carry):
            off, t = carry
            off = filter_append(buf, None, None, cidx,
                                c * CHUNK, (c + 1) * CHUNK, t, off, True)
            return lax.cond(off > CAP, compact, lambda x: x, (off, t))

        off, _t = lax.fori_loop(0, NITER // CHUNK, pb_chunk,
                                (jnp.int32(0), t0))

        gather_values(off)

        cval[pl.ds(off, 16)] = jnp.full((16,), NEG_INF)
        nv = (off + 15) // 16
        t1 = subset_min_threshold(cval, nv)

        def rf(j, o2):
            v = cval[pl.ds(j * 16, 16)]
            ix = cidx[pl.ds(j * 16, 16)]
            mask = v >= t1
            plsc.store_compressed(cval2.at[pl.ds(o2, 16)], v, mask=mask)
            plsc.store_compressed(cidx2.at[pl.ds(o2, 16)], ix, mask=mask)
            return o2 + jnp.sum(mask.astype(jnp.int32))

        off2 = lax.fori_loop(0, nv, rf, jnp.int32(0))

        extract_topk(cval2, cidx2, off2, r * K)
        return 0

    row0 = wid * RPW
    pltpu.async_copy(logits_hbm.at[row0], row_va, sem_a)

    def two_rows(q, _):
        r0 = 2 * q
        pltpu.make_async_copy(logits_hbm.at[row0 + r0], row_va, sem_a).wait()
        pltpu.async_copy(logits_hbm.at[row0 + r0 + 1], row_vb, sem_b)
        row_body(r0, row_va)
        pltpu.make_async_copy(
            logits_hbm.at[row0 + r0 + 1], row_vb, sem_b).wait()

        @pl.when(q + 1 < RPW // 2)
        def _():
            pltpu.async_copy(logits_hbm.at[row0 + r0 + 2], row_va, sem_a)

        row_body(r0 + 1, row_vb)
        return 0

    lax.fori_loop(0, RPW // 2, two_rows, 0)
    base = wid * (RPW * K)
    pltpu.sync_copy(outv.at[pl.ds(0, RPW * K)], vals_hbm.at[pl.ds(base, RPW * K)])
    pltpu.sync_copy(outi.at[pl.ds(0, RPW * K)], idx_hbm.at[pl.ds(base, RPW * K)])


def kernel(logits):
    mesh = plsc.VectorSubcoreMesh(core_axis_name="c", subcore_axis_name="s")
    f = functools.partial(
        pl.kernel,
        mesh=mesh,
        compiler_params=pltpu.CompilerParams(
            needs_layout_passes=False, use_tc_tiling_on_sc=True),
        out_type=[
            jax.ShapeDtypeStruct((ROWS * K,), jnp.float32),
            jax.ShapeDtypeStruct((ROWS * K,), jnp.int32),
        ],
        scratch_types=[
            pltpu.VMEM((COLS,), jnp.float32),
            pltpu.VMEM((COLS,), jnp.float32),
            pltpu.VMEM((BUF,), jnp.float32),
            pltpu.VMEM((BUF,), jnp.int32),
            pltpu.VMEM((BUF,), jnp.float32),
            pltpu.VMEM((BUF,), jnp.int32),
            pltpu.VMEM((RPW * K + K + 16,), jnp.float32),
            pltpu.VMEM((RPW * K + K + 16,), jnp.int32),
            pltpu.SemaphoreType.DMA,
            pltpu.SemaphoreType.DMA,
        ],
    )(_body)
    vals, idx = f(logits)
    return (vals.reshape(ROWS, K), idx.reshape(ROWS, K))


# speedup vs baseline: 1.0400x; 1.0400x over previous
import functools

import jax
import jax.numpy as jnp
from jax import lax
from jax.experimental import pallas as pl
from jax.experimental.pallas import tpu as pltpu
from jax.experimental.pallas import tpu_sc as plsc

K = 64
ROWS = 128
COLS = 32768
NITER = COLS // 16
CHUNK = 512
CAP = 4096
BUF = CAP + 16 * CHUNK + 16
BIG = 1 << 30
NEG_INF = float("-inf")

NC, NS = 2, 16
NW = NC * NS
RPW = ROWS // NW


def _body(logits_hbm, vals_hbm, idx_hbm,
          row_va, row_vb, cval, cidx, cval2, cidx2, outv, outi,
          sem_a, sem_b):
    lane = lax.iota(jnp.int32, 16)
    lane0 = lane == 0
    wid = lax.axis_index("s") * NC + lax.axis_index("c")

    def subset_min_threshold(src, nv):
        ninf = jnp.full((16,), NEG_INF)

        @plsc.parallel_loop(0, nv - (nv % 4), step=4, unroll=4,
                            carry=(ninf, ninf, ninf, ninf))
        def acc(j, c):
            a0, a1, a2, a3 = c
            a0 = jnp.maximum(a0, src[pl.ds(j * 16, 16)])
            a1 = jnp.maximum(a1, src[pl.ds((j + 1) * 16, 16)])
            a2 = jnp.maximum(a2, src[pl.ds((j + 2) * 16, 16)])
            a3 = jnp.maximum(a3, src[pl.ds((j + 3) * 16, 16)])
            return (a0, a1, a2, a3)

        a0, a1, a2, a3 = acc
        def leftover(j, a):
            return jnp.maximum(a, src[pl.ds(j * 16, 16)])

        a0 = lax.fori_loop(nv - (nv % 4), nv, leftover, a0)
        return jnp.min(jnp.minimum(jnp.minimum(a0, a1),
                                   jnp.minimum(a2, a3)))

    def filter_append(src_v, src_i, dval, didx, lo, hi, t, off, use_iota,
                      pipelined=True):
        offv = jnp.full((16,), jnp.int32(0)) + off

        def step(i, offv):
            v = src_v[pl.ds(i * 16, 16)]
            if use_iota:
                ix = i * 16 + lane
            else:
                ix = src_i[pl.ds(i * 16, 16)]
            mask = v >= t
            ones = mask.astype(jnp.int32)
            pos = offv + plsc.cumsum(ones) - 1
            if dval is not None:
                plsc.store_scatter(dval, [pos], v, mask=mask)
            plsc.store_scatter(didx, [pos], ix, mask=mask)
            return offv + plsc.all_reduce_population_count(mask)

        if pipelined:
            run = plsc.parallel_loop(lo, hi, unroll=4, carry=offv)(step)
        else:
            run = lax.fori_loop(lo, hi, step, offv)
        return jnp.max(run)

    def extract_topk(bval, bidx, off, obase=0):
        bval[pl.ds(off, 16)] = jnp.full((16,), NEG_INF)
        nv = (off + 15) // 16
        ninf = jnp.full((16,), NEG_INF)
        bigv = jnp.full((16,), jnp.int32(BIG))

        def ext_step(k, prev):
            m_prev, j_prev = prev

            @plsc.parallel_loop(0, nv, unroll=2, carry=(ninf, bigv))
            def scan(j, acc):
                av, ai = acc
                v = bval[pl.ds(j * 16, 16)]
                ix = bidx[pl.ds(j * 16, 16)]
                hit = (v == m_prev) & (ix == j_prev)
                v = jnp.where(hit, NEG_INF, v)
                bval[pl.ds(j * 16, 16)] = v
                take = (v > av) | ((v == av) & (ix < ai))
                return (jnp.where(take, v, av), jnp.where(take, ix, ai))

            av, ai = scan
            m = jnp.max(av)
            jm = jnp.min(jnp.where(av == m, ai, BIG))
            plsc.store_compressed(
                outv.at[pl.ds(obase + k, 16)], jnp.full((16,), m), mask=lane0)
            plsc.store_compressed(
                outi.at[pl.ds(obase + k, 16)], jnp.full((16,), jm), mask=lane0)
            return (m, jm)

        m, _ = lax.fori_loop(0, K, ext_step,
                             (jnp.float32(jnp.inf), jnp.int32(-1)))
        return m

    def row_body(r, buf):
        def gather_values(off):
            cidx[pl.ds(off, 16)] = jnp.full((16,), jnp.int32(0))
            nv = (off + 15) // 16

            def gv(j, _):
                ix = cidx[pl.ds(j * 16, 16)]
                cval[pl.ds(j * 16, 16)] = plsc.load_gather(buf, [ix])
                return 0

            lax.fori_loop(0, nv, gv, 0)

        def compact(carry):
            off, t = carry
            gather_values(off)
            t_new = extract_topk(cval, cidx, off, RPW * K)
            for j in range(K // 16):
                cidx[pl.ds(j * 16, 16)] = outi[pl.ds(RPW * K + j * 16, 16)]
            return (jnp.int32(K), t_new)

        t0 = subset_min_threshold(buf, NITER)

        def pb_chunk(c, carry):
            off, t = carry
            off = filter_append(buf, None, None, cidx,
                                c * CHUNK, (c + 1) * CHUNK, t, off, True)
            return lax.cond(off > CAP, compact, lambda x: x, (off, t))

        off, _t = lax.fori_loop(0, NITER // CHUNK, pb_chunk,
                                (jnp.int32(0), t0))

        gather_values(off)

        cval[pl.ds(off, 16)] = jnp.full((16,), NEG_INF)
        nv = (off + 15) // 16
        t1 = subset_min_threshold(cval, nv)

        def rf(j, o2):
            v = cval[pl.ds(j * 16, 16)]
            ix = cidx[pl.ds(j * 16, 16)]
            mask = v >= t1
            plsc.store_compressed(cval2.at[pl.ds(o2, 16)], v, mask=mask)
            plsc.store_compressed(cidx2.at[pl.ds(o2, 16)], ix, mask=mask)
            return o2 + jnp.sum(mask.astype(jnp.int32))

        off2 = lax.fori_loop(0, nv, rf, jnp.int32(0))

        extract_topk(cval2, cidx2, off2, r * K)
        return 0

    row0 = wid * RPW
    pltpu.async_copy(logits_hbm.at[row0], row_va, sem_a)

    def two_rows(q, _):
        r0 = 2 * q
        pltpu.make_async_copy(logits_hbm.at[row0 + r0], row_va, sem_a).wait()
        pltpu.async_copy(logits_hbm.at[row0 + r0 + 1], row_vb, sem_b)
        row_body(r0, row_va)
        pltpu.make_async_copy(
            logits_hbm.at[row0 + r0 + 1], row_vb, sem_b).wait()

        @pl.when(q + 1 < RPW // 2)
        def _():
            pltpu.async_copy(logits_hbm.at[row0 + r0 + 2], row_va, sem_a)

        row_body(r0 + 1, row_vb)
        return 0

    lax.fori_loop(0, RPW // 2, two_rows, 0)
    base = wid * (RPW * K)
    pltpu.sync_copy(outv.at[pl.ds(0, RPW * K)], vals_hbm.at[pl.ds(base, RPW * K)])
    pltpu.sync_copy(outi.at[pl.ds(0, RPW * K)], idx_hbm.at[pl.ds(base, RPW * K)])


def kernel(logits):
    mesh = plsc.VectorSubcoreMesh(core_axis_name="c", subcore_axis_name="s")
    f = functools.partial(
        pl.kernel,
        mesh=mesh,
        compiler_params=pltpu.CompilerParams(
            needs_layout_passes=False, use_tc_tiling_on_sc=True),
        out_type=[
            jax.ShapeDtypeStruct((ROWS * K,), jnp.float32),
            jax.ShapeDtypeStruct((ROWS * K,), jnp.int32),
        ],
        scratch_types=[
            pltpu.VMEM((COLS,), jnp.float32),
            pltpu.VMEM((COLS,), jnp.float32),
            pltpu.VMEM((BUF,), jnp.float32),
            pltpu.VMEM((BUF,), jnp.int32),
            pltpu.VMEM((BUF,), jnp.float32),
            pltpu.VMEM((BUF,), jnp.int32),
            pltpu.VMEM((RPW * K + K + 16,), jnp.float32),
            pltpu.VMEM((RPW * K + K + 16,), jnp.int32),
            pltpu.SemaphoreType.DMA,
            pltpu.SemaphoreType.DMA,
        ],
    )(_body)
    vals, idx = f(logits)
    return (vals.reshape(ROWS, K), idx.reshape(ROWS, K))
